# baseline (speedup 1.0000x reference)
import functools

import jax
import jax.numpy as jnp
from jax import lax
from jax.experimental import pallas as pl
from jax.experimental.pallas import tpu as pltpu
from jax.experimental.pallas import tpu_sc as plsc

N = 10000
E = 320000
D = 128
OUT = 64
NC = 2
NS = 16
NW = NC * NS
K = 128
CHUNKS_PAD = 2560
E_PAD = CHUNKS_PAD * K
NA = N + 8
TROWS = 640
TCHUNKS = TROWS // K


def _tile_start(s):
    return jnp.minimum(s * 632, NA - TROWS)


def _zero_acc(s, zrows, rows, acc):
    st = _tile_start(s)
    pltpu.sync_copy(zrows, rows)
    for i in range(TCHUNKS):
        pltpu.sync_copy(rows, acc.at[pl.ds(st + i * K, K)])
    return st


def _copy_out(c, st, acc, rows, out):
    for i in range(TCHUNKS):
        pltpu.sync_copy(acc.at[pl.ds(st + i * K, K)], rows)
        pltpu.sync_copy(rows, out.at[c, pl.ds(st + i * K, K)])


def _mk_cnt_agg():
    mesh = plsc.VectorSubcoreMesh(core_axis_name="c", subcore_axis_name="s")
    CPT = CHUNKS_PAD // NS

    @functools.partial(
        pl.kernel,
        out_type=jax.ShapeDtypeStruct((NC, NA, D), jnp.float32),
        mesh=mesh,
        scratch_types=[
            pltpu.VMEM((CPT, K), jnp.int32),
            pltpu.VMEM((K, D), jnp.float32),
            pltpu.VMEM_SHARED((NA, D), jnp.float32),
            pltpu.SemaphoreType.DMA,
        ],
    )
    def k(edst2, ones_hbm, zrows, out_cnt, didx, rows, acc, sem):
        c = lax.axis_index("c")
        s = lax.axis_index("s")
        st = _zero_acc(s, zrows, rows, acc)
        pltpu.sync_copy(ones_hbm, rows)
        pltpu.sync_copy(edst2.at[pl.ds(c * CHUNKS_PAD + s * CPT, CPT)],
                        didx)
        plsc.subcore_barrier()

        pltpu.async_copy(rows, acc.at[didx.at[0]], sem, add=True)

        def body(j, carry):
            pltpu.async_copy(rows, acc.at[didx.at[j]], sem, add=True)
            pltpu.make_async_copy(rows, acc.at[didx.at[0]], sem).wait()
            return carry

        lax.fori_loop(1, CPT, body, 0)
        pltpu.make_async_copy(rows, acc.at[didx.at[0]], sem).wait()
        plsc.subcore_barrier()
        _copy_out(c, st, acc, rows, out_cnt)

    return k


def _mk_feat_agg(tiles_per_list):
    mesh = plsc.VectorSubcoreMesh(core_axis_name="c", subcore_axis_name="s")
    CPT = CHUNKS_PAD // tiles_per_list
    IB = 32 if tiles_per_list == NS else 16
    NB = CPT // IB

    @functools.partial(
        pl.kernel,
        out_type=jax.ShapeDtypeStruct((NC, NA, D), jnp.float32),
        mesh=mesh,
        scratch_types=[
            pltpu.VMEM((IB * K,), jnp.int32),
            pltpu.VMEM((IB, K), jnp.int32),
            pltpu.VMEM((K, D), jnp.float32),
            pltpu.VMEM((K, D), jnp.float32),
            pltpu.VMEM_SHARED((NA, D), jnp.float32),
            pltpu.SemaphoreType.DMA,
            pltpu.SemaphoreType.DMA,
            pltpu.SemaphoreType.DMA,
            pltpu.SemaphoreType.DMA,
        ],
    )
    def k(x, esrc, edst2, zrows, out_sum, sidx, didx, rows_a, rows_b, acc,
          sem_a, sem_b, sem_sa, sem_sb):
        c = lax.axis_index("c")
        s = lax.axis_index("s")
        if tiles_per_list == NS:
            tid = s
            ebase = c * E_PAD
            rbase = c * CHUNKS_PAD
        else:
            tid = s * NC + c
            ebase = 0
            rbase = 0
        st = _zero_acc(s, zrows, rows_a, acc)
        plsc.subcore_barrier()

        def sl(j):
            return sidx.at[pl.ds(j * K, K)]

        def wait(buf, sem):
            pltpu.make_async_copy(x.at[sl(0)], buf, sem).wait()

        def wait_s(buf, sem):
            pltpu.make_async_copy(buf, acc.at[didx.at[0]], sem).wait()

        def blk(b, carry):
            pltpu.sync_copy(
                esrc.at[pl.ds(ebase + (tid * CPT + b * IB) * K, IB * K)],
                sidx)
            pltpu.sync_copy(
                edst2.at[pl.ds(rbase + tid * CPT + b * IB, IB)], didx)
            pltpu.async_copy(x.at[sl(0)], rows_a, sem_a)
            pltpu.async_copy(x.at[sl(1)], rows_b, sem_b)

            def body(i, carry2):
                j0 = 2 * i
                wait(rows_a, sem_a)
                pltpu.async_copy(rows_a, acc.at[didx.at[j0]], sem_sa,
                                 add=True)
                wait(rows_b, sem_b)
                pltpu.async_copy(rows_b, acc.at[didx.at[j0 + 1]], sem_sb,
                                 add=True)
                wait_s(rows_a, sem_sa)
                pltpu.async_copy(x.at[sl(jnp.minimum(j0 + 2, IB - 1))],
                                 rows_a, sem_a)
                wait_s(rows_b, sem_sb)
                pltpu.async_copy(x.at[sl(jnp.minimum(j0 + 3, IB - 1))],
                                 rows_b, sem_b)
                return carry2

            lax.fori_loop(0, IB // 2, body, 0)
            wait(rows_a, sem_a)
            wait(rows_b, sem_b)
            return carry

        lax.fori_loop(0, NB, blk, 0)
        plsc.subcore_barrier()
        _copy_out(c, st, acc, rows_a, out_sum)

    return k


_cnt_agg = _mk_cnt_agg()
_l1_agg = _mk_feat_agg(NS)
_l2_agg = _mk_feat_agg(NW)

_MB = 1000


def _lrelu(x):
    return jnp.where(x >= 0, x, 0.01 * x)


def _dense1_body(sum_st, cnt_st, xt, sum_ts, cnt_ts, xs,
                 wstn, wstr, bst, wtsn, wtsr, bts, xs1_o, xt1_o):
    mean_t = sum_st[...] / jnp.maximum(cnt_st[:, 0:1], 1.0)
    t1 = (jnp.dot(mean_t, wstn[...], preferred_element_type=jnp.float32)
          + jnp.dot(xt[...], wstr[...], preferred_element_type=jnp.float32)
          + bst[...])
    xt1_o[...] = _lrelu(t1)
    mean_s = sum_ts[...] / jnp.maximum(cnt_ts[:, 0:1], 1.0)
    s1 = (jnp.dot(mean_s, wtsn[...], preferred_element_type=jnp.float32)
          + jnp.dot(xs[...], wtsr[...], preferred_element_type=jnp.float32)
          + bts[...])
    xs1_o[...] = _lrelu(s1)


def _dense2_body(p0, p1, cnt_ts, xs1, wn, wr, b2, wlin, blin, out_o):
    mean2 = (p0[...] + p1[...]) / jnp.maximum(cnt_ts[:, 0:1], 1.0)
    s2 = (jnp.dot(mean2, wn[...], preferred_element_type=jnp.float32)
          + jnp.dot(xs1[...], wr[...], preferred_element_type=jnp.float32)
          + b2[...])
    out_o[...] = (jnp.dot(_lrelu(s2), wlin[...],
                          preferred_element_type=jnp.float32) + blin[...])


def _row_spec(w):
    return pl.BlockSpec((_MB, w), lambda i: (i, 0))


def _full_spec(shape):
    nd = len(shape)
    return pl.BlockSpec(shape, lambda i: (0,) * nd)


def _dense1(sum_st, cnt_st, xt, sum_ts, cnt_ts, xs,
            wstn, wstr, bst, wtsn, wtsr, bts):
    grid = (N // _MB,)
    return pl.pallas_call(
        _dense1_body,
        grid=grid,
        in_specs=[
            _row_spec(D), _row_spec(D), _row_spec(D),
            _row_spec(D), _row_spec(D), _row_spec(D),
            _full_spec((D, D)), _full_spec((D, D)), _full_spec((1, D)),
            _full_spec((D, D)), _full_spec((D, D)), _full_spec((1, D)),
        ],
        out_specs=[_row_spec(D), _row_spec(D)],
        out_shape=[jax.ShapeDtypeStruct((N, D), jnp.float32),
                   jax.ShapeDtypeStruct((N, D), jnp.float32)],
    )(sum_st, cnt_st, xt, sum_ts, cnt_ts, xs,
      wstn, wstr, bst.reshape(1, D), wtsn, wtsr, bts.reshape(1, D))


def _dense2(p0, p1, cnt_ts, xs1, wn, wr, b2, wlin, blin):
    grid = (N // _MB,)
    return pl.pallas_call(
        _dense2_body,
        grid=grid,
        in_specs=[
            _row_spec(D), _row_spec(D), _row_spec(D), _row_spec(D),
            _full_spec((D, D)), _full_spec((D, D)), _full_spec((1, D)),
            _full_spec((D, OUT)), _full_spec((1, OUT)),
        ],
        out_specs=_row_spec(OUT),
        out_shape=jax.ShapeDtypeStruct((N, OUT), jnp.float32),
    )(p0, p1, cnt_ts, xs1, wn, wr, b2.reshape(1, D), wlin,
      blin.reshape(1, OUT))


def kernel(x_source, x_target, W1_st_n, W1_st_r, b1_st, W1_ts_n, W1_ts_r,
           b1_ts, W2_st_n, W2_st_r, b2_st, W2_ts_n, W2_ts_r, b2_ts,
           W_lin, b_lin, edge_index_st, edge_index_ts):
    del W2_st_n, W2_st_r, b2_st
    ei_st = edge_index_st.astype(jnp.int32)
    ei_ts = edge_index_ts.astype(jnp.int32)

    pad_s = jnp.zeros((E_PAD - E,), jnp.int32)
    pad_d = jnp.full((E_PAD - E,), N, jnp.int32)
    xcat = jnp.concatenate([x_source, x_target], axis=0)
    esrc = jnp.concatenate([ei_st[0], pad_s, ei_ts[0] + N, pad_s])
    edst = jnp.concatenate([ei_st[1], pad_d, ei_ts[1], pad_d])
    ones_rows = jnp.ones((K, D), jnp.float32)
    zrows = jnp.zeros((K, D), jnp.float32)

    edst2 = edst.reshape(CHUNKS_PAD * NC, K)
    cnts = _cnt_agg(edst2, ones_rows, zrows)
    sums = _l1_agg(xcat, esrc, edst2, zrows)

    xs1, xt1 = _dense1(sums[0, :N], cnts[0, :N], x_target, sums[1, :N],
                       cnts[1, :N], x_source, W1_st_n, W1_st_r, b1_st,
                       W1_ts_n, W1_ts_r, b1_ts)

    p = _l2_agg(xt1, jnp.concatenate([ei_ts[0], pad_s]),
                jnp.concatenate([ei_ts[1], pad_d]).reshape(CHUNKS_PAD, K),
                zrows)
    return _dense2(p[0, :N], p[1, :N], cnts[1, :N], xs1, W2_ts_n, W2_ts_r,
                   b2_ts, W_lin, b_lin)


# speedup vs baseline: 1.0832x; 1.0832x over previous
import functools

import jax
import jax.numpy as jnp
from jax import lax
from jax.experimental import pallas as pl
from jax.experimental.pallas import tpu as pltpu
from jax.experimental.pallas import tpu_sc as plsc

N = 10000
E = 320000
D = 128
OUT = 64
NC = 2
NS = 16
NW = NC * NS
K = 128
CHUNKS_PAD = 2560
E_PAD = CHUNKS_PAD * K
NA = N + 8
TROWS = 640
TCHUNKS = TROWS // K


def _tile_start(s):
    return jnp.minimum(s * 632, NA - TROWS)


def _zero_acc(s, zrows, rows, acc):
    st = _tile_start(s)
    pltpu.sync_copy(zrows, rows)
    for i in range(TCHUNKS):
        pltpu.sync_copy(rows, acc.at[pl.ds(st + i * K, K)])
    return st


def _copy_out(c, st, acc, rows, out):
    for i in range(TCHUNKS):
        pltpu.sync_copy(acc.at[pl.ds(st + i * K, K)], rows)
        pltpu.sync_copy(rows, out.at[c, pl.ds(st + i * K, K)])


def _mk_cnt_agg():
    mesh = plsc.VectorSubcoreMesh(core_axis_name="c", subcore_axis_name="s")
    CPT = CHUNKS_PAD // NS

    @functools.partial(
        pl.kernel,
        out_type=jax.ShapeDtypeStruct((NC, NA, D), jnp.float32),
        mesh=mesh,
        scratch_types=[
            pltpu.VMEM((CPT, K), jnp.int32),
            pltpu.VMEM((K, D), jnp.float32),
            pltpu.VMEM_SHARED((NA, D), jnp.float32),
            pltpu.SemaphoreType.DMA,
        ],
    )
    def k(edst2, ones_hbm, zrows, out_cnt, didx, rows, acc, sem):
        c = lax.axis_index("c")
        s = lax.axis_index("s")
        st = _zero_acc(s, zrows, rows, acc)
        pltpu.sync_copy(ones_hbm, rows)
        pltpu.sync_copy(edst2.at[pl.ds(c * CHUNKS_PAD + s * CPT, CPT)],
                        didx)
        plsc.subcore_barrier()

        pltpu.async_copy(rows, acc.at[didx.at[0]], sem, add=True)

        def body(j, carry):
            pltpu.async_copy(rows, acc.at[didx.at[j]], sem, add=True)
            pltpu.make_async_copy(rows, acc.at[didx.at[0]], sem).wait()
            return carry

        lax.fori_loop(1, CPT, body, 0)
        pltpu.make_async_copy(rows, acc.at[didx.at[0]], sem).wait()
        plsc.subcore_barrier()
        _copy_out(c, st, acc, rows, out_cnt)

    return k


def _mk_feat_agg(tiles_per_list):
    mesh = plsc.VectorSubcoreMesh(core_axis_name="c", subcore_axis_name="s")
    CPT = CHUNKS_PAD // tiles_per_list
    IB = 32 if tiles_per_list == NS else 16
    NB = CPT // IB

    @functools.partial(
        pl.kernel,
        out_type=jax.ShapeDtypeStruct((NC, NA, D), jnp.float32),
        mesh=mesh,
        scratch_types=[
            pltpu.VMEM((IB * K,), jnp.int32),
            pltpu.VMEM((IB, K), jnp.int32),
            pltpu.VMEM((2 * K, D), jnp.float32),
            pltpu.VMEM_SHARED((NA, D), jnp.float32),
            pltpu.SemaphoreType.DMA,
            pltpu.SemaphoreType.DMA,
            pltpu.SemaphoreType.DMA,
            pltpu.SemaphoreType.DMA,
            pltpu.SemaphoreType.DMA,
            pltpu.SemaphoreType.DMA,
        ],
    )
    def k(x, esrc, edst2, zrows, out_sum, sidx, didx, rows, acc,
          sem_g0, sem_g1, sem_g2, sem_g3, sem_sa, sem_sb):
        c = lax.axis_index("c")
        s = lax.axis_index("s")
        if tiles_per_list == NS:
            tid = s
            ebase = c * E_PAD
            rbase = c * CHUNKS_PAD
        else:
            tid = s * NC + c
            ebase = 0
            rbase = 0
        rows_a = rows.at[pl.ds(0, K)]
        rows_b = rows.at[pl.ds(K, K)]
        st = _zero_acc(s, zrows, rows_a, acc)
        plsc.subcore_barrier()

        def sl(j):
            return sidx.at[pl.ds(j * K, K)]

        def wait(buf, sem):
            pltpu.make_async_copy(x.at[sl(0)], buf, sem).wait()

        def blk(b, carry):
            pltpu.sync_copy(
                esrc.at[pl.ds(ebase + (tid * CPT + b * IB) * K, IB * K)],
                sidx)
            pltpu.sync_copy(
                edst2.at[pl.ds(rbase + tid * CPT + b * IB, IB)], didx)
            pltpu.async_copy(x.at[sl(0)], rows_a, sem_g0)

            def body(i, carry2):
                j0 = 2 * i
                pltpu.async_copy(x.at[sl(j0 + 1)], rows_b, sem_g1,
                                 priority=1)
                wait(rows_a, sem_g0)
                pltpu.sync_copy(rows_a, acc.at[didx.at[j0]], add=True)
                pltpu.async_copy(x.at[sl(jnp.minimum(j0 + 2, IB - 1))],
                                 rows_a, sem_g0)
                wait(rows_b, sem_g1)
                pltpu.sync_copy(rows_b, acc.at[didx.at[j0 + 1]], add=True)
                return carry2

            lax.fori_loop(0, IB // 2, body, 0)
            wait(rows_a, sem_g0)
            return carry

        lax.fori_loop(0, NB, blk, 0)
        plsc.subcore_barrier()
        _copy_out(c, st, acc, rows_a, out_sum)

    return k


_cnt_agg = _mk_cnt_agg()
_l1_agg = _mk_feat_agg(NS)
_l2_agg = _mk_feat_agg(NW)

_MB = 1000


def _lrelu(x):
    return jnp.where(x >= 0, x, 0.01 * x)


def _dense1_body(sum_st, cnt_st, xt, sum_ts, cnt_ts, xs,
                 wstn, wstr, bst, wtsn, wtsr, bts, xs1_o, xt1_o):
    mean_t = sum_st[...] / jnp.maximum(cnt_st[:, 0:1], 1.0)
    t1 = (jnp.dot(mean_t, wstn[...], preferred_element_type=jnp.float32)
          + jnp.dot(xt[...], wstr[...], preferred_element_type=jnp.float32)
          + bst[...])
    xt1_o[...] = _lrelu(t1)
    mean_s = sum_ts[...] / jnp.maximum(cnt_ts[:, 0:1], 1.0)
    s1 = (jnp.dot(mean_s, wtsn[...], preferred_element_type=jnp.float32)
          + jnp.dot(xs[...], wtsr[...], preferred_element_type=jnp.float32)
          + bts[...])
    xs1_o[...] = _lrelu(s1)


def _dense2_body(p0, p1, cnt_ts, xs1, wn, wr, b2, wlin, blin, out_o):
    mean2 = (p0[...] + p1[...]) / jnp.maximum(cnt_ts[:, 0:1], 1.0)
    s2 = (jnp.dot(mean2, wn[...], preferred_element_type=jnp.float32)
          + jnp.dot(xs1[...], wr[...], preferred_element_type=jnp.float32)
          + b2[...])
    out_o[...] = (jnp.dot(_lrelu(s2), wlin[...],
                          preferred_element_type=jnp.float32) + blin[...])


def _row_spec(w):
    return pl.BlockSpec((_MB, w), lambda i: (i, 0))


def _full_spec(shape):
    nd = len(shape)
    return pl.BlockSpec(shape, lambda i: (0,) * nd)


def _dense1(sum_st, cnt_st, xt, sum_ts, cnt_ts, xs,
            wstn, wstr, bst, wtsn, wtsr, bts):
    grid = (N // _MB,)
    return pl.pallas_call(
        _dense1_body,
        grid=grid,
        in_specs=[
            _row_spec(D), _row_spec(D), _row_spec(D),
            _row_spec(D), _row_spec(D), _row_spec(D),
            _full_spec((D, D)), _full_spec((D, D)), _full_spec((1, D)),
            _full_spec((D, D)), _full_spec((D, D)), _full_spec((1, D)),
        ],
        out_specs=[_row_spec(D), _row_spec(D)],
        out_shape=[jax.ShapeDtypeStruct((N, D), jnp.float32),
                   jax.ShapeDtypeStruct((N, D), jnp.float32)],
    )(sum_st, cnt_st, xt, sum_ts, cnt_ts, xs,
      wstn, wstr, bst.reshape(1, D), wtsn, wtsr, bts.reshape(1, D))


def _dense2(p0, p1, cnt_ts, xs1, wn, wr, b2, wlin, blin):
    grid = (N // _MB,)
    return pl.pallas_call(
        _dense2_body,
        grid=grid,
        in_specs=[
            _row_spec(D), _row_spec(D), _row_spec(D), _row_spec(D),
            _full_spec((D, D)), _full_spec((D, D)), _full_spec((1, D)),
            _full_spec((D, OUT)), _full_spec((1, OUT)),
        ],
        out_specs=_row_spec(OUT),
        out_shape=jax.ShapeDtypeStruct((N, OUT), jnp.float32),
    )(p0, p1, cnt_ts, xs1, wn, wr, b2.reshape(1, D), wlin,
      blin.reshape(1, OUT))


def kernel(x_source, x_target, W1_st_n, W1_st_r, b1_st, W1_ts_n, W1_ts_r,
           b1_ts, W2_st_n, W2_st_r, b2_st, W2_ts_n, W2_ts_r, b2_ts,
           W_lin, b_lin, edge_index_st, edge_index_ts):
    del W2_st_n, W2_st_r, b2_st
    ei_st = edge_index_st.astype(jnp.int32)
    ei_ts = edge_index_ts.astype(jnp.int32)

    pad_s = jnp.zeros((E_PAD - E,), jnp.int32)
    pad_d = jnp.full((E_PAD - E,), N, jnp.int32)
    xcat = jnp.concatenate([x_source, x_target], axis=0)
    esrc = jnp.concatenate([ei_st[0], pad_s, ei_ts[0] + N, pad_s])
    edst = jnp.concatenate([ei_st[1], pad_d, ei_ts[1], pad_d])
    ones_rows = jnp.ones((K, D), jnp.float32)
    zrows = jnp.zeros((K, D), jnp.float32)

    edst2 = edst.reshape(CHUNKS_PAD * NC, K)
    cnts = _cnt_agg(edst2, ones_rows, zrows)
    sums = _l1_agg(xcat, esrc, edst2, zrows)

    xs1, xt1 = _dense1(sums[0, :N], cnts[0, :N], x_target, sums[1, :N],
                       cnts[1, :N], x_source, W1_st_n, W1_st_r, b1_st,
                       W1_ts_n, W1_ts_r, b1_ts)

    p = _l2_agg(xt1, jnp.concatenate([ei_ts[0], pad_s]),
                jnp.concatenate([ei_ts[1], pad_d]).reshape(CHUNKS_PAD, K),
                zrows)
    return _dense2(p[0, :N], p[1, :N], cnts[1, :N], xs1, W2_ts_n, W2_ts_r,
                   b2_ts, W_lin, b_lin)


# speedup vs baseline: 1.0856x; 1.0021x over previous
import functools

import jax
import jax.numpy as jnp
from jax import lax
from jax.experimental import pallas as pl
from jax.experimental.pallas import tpu as pltpu
from jax.experimental.pallas import tpu_sc as plsc

N = 10000
E = 320000
D = 128
OUT = 64
NC = 2
NS = 16
NW = NC * NS
K = 128
CHUNKS_PAD = 2560
E_PAD = CHUNKS_PAD * K
NA = N + 8
TROWS = 640
TCHUNKS = TROWS // K


def _tile_start(s):
    return jnp.minimum(s * 632, NA - TROWS)


def _zero_acc(s, zrows, rows, acc, sem):
    st = _tile_start(s)
    pltpu.sync_copy(zrows, rows)
    for i in range(TCHUNKS):
        pltpu.async_copy(rows, acc.at[pl.ds(st + i * K, K)], sem)
    for i in range(TCHUNKS):
        pltpu.make_async_copy(rows, acc.at[pl.ds(st, K)], sem).wait()
    return st


def _copy_out(c, st, acc, bufs, sems, out):
    pltpu.sync_copy(acc.at[pl.ds(st, K)], bufs[0])
    for i in range(TCHUNKS):
        cur, sem = bufs[i % 2], sems[i % 2]
        pltpu.async_copy(cur, out.at[c, pl.ds(st + i * K, K)], sem)
        if i + 1 < TCHUNKS:
            nxt, nsem = bufs[(i + 1) % 2], sems[(i + 1) % 2]
            if i >= 1:
                pltpu.make_async_copy(nxt, out.at[c, pl.ds(st, K)],
                                      nsem).wait()
            pltpu.sync_copy(acc.at[pl.ds(st + (i + 1) * K, K)], nxt)
    for q in range(2):
        pltpu.make_async_copy(bufs[q], out.at[c, pl.ds(st, K)],
                              sems[q]).wait()


def _mk_cnt_agg():
    mesh = plsc.VectorSubcoreMesh(core_axis_name="c", subcore_axis_name="s")
    CPT = CHUNKS_PAD // NS
    IBC = 32

    @functools.partial(
        pl.kernel,
        out_type=jax.ShapeDtypeStruct((NC, NA, D), jnp.float32),
        mesh=mesh,
        scratch_types=[
            pltpu.VMEM((IBC, K), jnp.int32),
            pltpu.VMEM((K, D), jnp.float32),
            pltpu.VMEM((K, D), jnp.float32),
            pltpu.VMEM_SHARED((NA, D), jnp.float32),
            pltpu.SemaphoreType.DMA,
            pltpu.SemaphoreType.DMA,
        ],
    )
    def k(edst2, ones_hbm, zrows, out_cnt, didx, rows_a, rows_b, acc,
          sem_a, sem_b):
        c = lax.axis_index("c")
        s = lax.axis_index("s")
        st = _zero_acc(s, zrows, rows_a, acc, sem_a)
        pltpu.sync_copy(ones_hbm, rows_a)
        plsc.subcore_barrier()

        def blk(b, carry):
            pltpu.sync_copy(
                edst2.at[pl.ds(c * CHUNKS_PAD + s * CPT + b * IBC, IBC)],
                didx)
            pltpu.async_copy(rows_a, acc.at[didx.at[0]], sem_a, add=True)

            def body(j, carry2):
                pltpu.async_copy(rows_a, acc.at[didx.at[j]], sem_a,
                                 add=True)
                pltpu.make_async_copy(rows_a, acc.at[didx.at[0]],
                                      sem_a).wait()
                return carry2

            lax.fori_loop(1, IBC, body, 0)
            pltpu.make_async_copy(rows_a, acc.at[didx.at[0]], sem_a).wait()
            return carry

        lax.fori_loop(0, CPT // IBC, blk, 0)
        plsc.subcore_barrier()
        _copy_out(c, st, acc, (rows_a, rows_b), (sem_a, sem_b), out_cnt)

    return k


def _mk_feat_agg(tiles_per_list):
    mesh = plsc.VectorSubcoreMesh(core_axis_name="c", subcore_axis_name="s")
    CPT = CHUNKS_PAD // tiles_per_list
    IB = 32 if tiles_per_list == NS else 16
    NB = CPT // IB

    @functools.partial(
        pl.kernel,
        out_type=jax.ShapeDtypeStruct((NC, NA, D), jnp.float32),
        mesh=mesh,
        scratch_types=[
            pltpu.VMEM((IB * K,), jnp.int32),
            pltpu.VMEM((IB, K), jnp.int32),
            pltpu.VMEM((2 * K, D), jnp.float32),
            pltpu.VMEM_SHARED((NA, D), jnp.float32),
            pltpu.SemaphoreType.DMA,
            pltpu.SemaphoreType.DMA,
            pltpu.SemaphoreType.DMA,
            pltpu.SemaphoreType.DMA,
            pltpu.SemaphoreType.DMA,
            pltpu.SemaphoreType.DMA,
        ],
    )
    def k(x, esrc, edst2, zrows, out_sum, sidx, didx, rows, acc,
          sem_g0, sem_g1, sem_g2, sem_g3, sem_sa, sem_sb):
        c = lax.axis_index("c")
        s = lax.axis_index("s")
        if tiles_per_list == NS:
            tid = s
            ebase = c * E_PAD
            rbase = c * CHUNKS_PAD
        else:
            tid = s * NC + c
            ebase = 0
            rbase = 0
        rows_a = rows.at[pl.ds(0, K)]
        rows_b = rows.at[pl.ds(K, K)]
        st = _zero_acc(s, zrows, rows_a, acc, sem_g0)
        plsc.subcore_barrier()

        def sl(j):
            return sidx.at[pl.ds(j * K, K)]

        def wait(buf, sem):
            pltpu.make_async_copy(x.at[sl(0)], buf, sem).wait()

        def blk(b, carry):
            pltpu.sync_copy(
                esrc.at[pl.ds(ebase + (tid * CPT + b * IB) * K, IB * K)],
                sidx)
            pltpu.sync_copy(
                edst2.at[pl.ds(rbase + tid * CPT + b * IB, IB)], didx)
            pltpu.async_copy(x.at[sl(0)], rows_a, sem_g0)

            def body(i, carry2):
                j0 = 2 * i
                pltpu.async_copy(x.at[sl(j0 + 1)], rows_b, sem_g1,
                                 priority=1)
                wait(rows_a, sem_g0)
                pltpu.sync_copy(rows_a, acc.at[didx.at[j0]], add=True)
                pltpu.async_copy(x.at[sl(jnp.minimum(j0 + 2, IB - 1))],
                                 rows_a, sem_g0)
                wait(rows_b, sem_g1)
                pltpu.sync_copy(rows_b, acc.at[didx.at[j0 + 1]], add=True)
                return carry2

            lax.fori_loop(0, IB // 2, body, 0)
            wait(rows_a, sem_g0)
            return carry

        lax.fori_loop(0, NB, blk, 0)
        plsc.subcore_barrier()
        _copy_out(c, st, acc, (rows_a, rows_b), (sem_g0, sem_g1), out_sum)

    return k


_cnt_agg = _mk_cnt_agg()
_l1_agg = _mk_feat_agg(NS)
_l2_agg = _mk_feat_agg(NW)

_MB = 1000


def _lrelu(x):
    return jnp.where(x >= 0, x, 0.01 * x)


def _dense1_body(sum_st, cnt_st, xt, sum_ts, cnt_ts, xs,
                 wstn, wstr, bst, wtsn, wtsr, bts, xs1_o, xt1_o):
    mean_t = sum_st[...] / jnp.maximum(cnt_st[:, 0:1], 1.0)
    t1 = (jnp.dot(mean_t, wstn[...], preferred_element_type=jnp.float32)
          + jnp.dot(xt[...], wstr[...], preferred_element_type=jnp.float32)
          + bst[...])
    xt1_o[...] = _lrelu(t1)
    mean_s = sum_ts[...] / jnp.maximum(cnt_ts[:, 0:1], 1.0)
    s1 = (jnp.dot(mean_s, wtsn[...], preferred_element_type=jnp.float32)
          + jnp.dot(xs[...], wtsr[...], preferred_element_type=jnp.float32)
          + bts[...])
    xs1_o[...] = _lrelu(s1)


def _dense2_body(p0, p1, cnt_ts, xs1, wn, wr, b2, wlin, blin, out_o):
    mean2 = (p0[...] + p1[...]) / jnp.maximum(cnt_ts[:, 0:1], 1.0)
    s2 = (jnp.dot(mean2, wn[...], preferred_element_type=jnp.float32)
          + jnp.dot(xs1[...], wr[...], preferred_element_type=jnp.float32)
          + b2[...])
    out_o[...] = (jnp.dot(_lrelu(s2), wlin[...],
                          preferred_element_type=jnp.float32) + blin[...])


def _row_spec(w):
    return pl.BlockSpec((_MB, w), lambda i: (i, 0))


def _full_spec(shape):
    nd = len(shape)
    return pl.BlockSpec(shape, lambda i: (0,) * nd)


def _dense1(sum_st, cnt_st, xt, sum_ts, cnt_ts, xs,
            wstn, wstr, bst, wtsn, wtsr, bts):
    grid = (N // _MB,)
    return pl.pallas_call(
        _dense1_body,
        grid=grid,
        in_specs=[
            _row_spec(D), _row_spec(D), _row_spec(D),
            _row_spec(D), _row_spec(D), _row_spec(D),
            _full_spec((D, D)), _full_spec((D, D)), _full_spec((1, D)),
            _full_spec((D, D)), _full_spec((D, D)), _full_spec((1, D)),
        ],
        out_specs=[_row_spec(D), _row_spec(D)],
        out_shape=[jax.ShapeDtypeStruct((N, D), jnp.float32),
                   jax.ShapeDtypeStruct((N, D), jnp.float32)],
    )(sum_st, cnt_st, xt, sum_ts, cnt_ts, xs,
      wstn, wstr, bst.reshape(1, D), wtsn, wtsr, bts.reshape(1, D))


def _dense2(p0, p1, cnt_ts, xs1, wn, wr, b2, wlin, blin):
    grid = (N // _MB,)
    return pl.pallas_call(
        _dense2_body,
        grid=grid,
        in_specs=[
            _row_spec(D), _row_spec(D), _row_spec(D), _row_spec(D),
            _full_spec((D, D)), _full_spec((D, D)), _full_spec((1, D)),
            _full_spec((D, OUT)), _full_spec((1, OUT)),
        ],
        out_specs=_row_spec(OUT),
        out_shape=jax.ShapeDtypeStruct((N, OUT), jnp.float32),
    )(p0, p1, cnt_ts, xs1, wn, wr, b2.reshape(1, D), wlin,
      blin.reshape(1, OUT))


def kernel(x_source, x_target, W1_st_n, W1_st_r, b1_st, W1_ts_n, W1_ts_r,
           b1_ts, W2_st_n, W2_st_r, b2_st, W2_ts_n, W2_ts_r, b2_ts,
           W_lin, b_lin, edge_index_st, edge_index_ts):
    del W2_st_n, W2_st_r, b2_st
    ei_st = edge_index_st.astype(jnp.int32)
    ei_ts = edge_index_ts.astype(jnp.int32)

    pad_s = jnp.zeros((E_PAD - E,), jnp.int32)
    pad_d = jnp.full((E_PAD - E,), N, jnp.int32)
    xcat = jnp.concatenate([x_source, x_target], axis=0)
    esrc = jnp.concatenate([ei_st[0], pad_s, ei_ts[0] + N, pad_s])
    edst = jnp.concatenate([ei_st[1], pad_d, ei_ts[1], pad_d])
    ones_rows = jnp.ones((K, D), jnp.float32)
    zrows = jnp.zeros((K, D), jnp.float32)

    edst2 = edst.reshape(CHUNKS_PAD * NC, K)
    cnts = _cnt_agg(edst2, ones_rows, zrows)
    sums = _l1_agg(xcat, esrc, edst2, zrows)

    xs1, xt1 = _dense1(sums[0, :N], cnts[0, :N], x_target, sums[1, :N],
                       cnts[1, :N], x_source, W1_st_n, W1_st_r, b1_st,
                       W1_ts_n, W1_ts_r, b1_ts)

    p = _l2_agg(xt1, jnp.concatenate([ei_ts[0], pad_s]),
                jnp.concatenate([ei_ts[1], pad_d]).reshape(CHUNKS_PAD, K),
                zrows)
    return _dense2(p[0, :N], p[1, :N], cnts[1, :N], xs1, W2_ts_n, W2_ts_r,
                   b2_ts, W_lin, b_lin)
